# baseline (speedup 1.0000x reference)
import functools

import jax
import jax.numpy as jnp
from jax import lax
from jax.experimental import pallas as pl
from jax.experimental.pallas import tpu as pltpu
from jax.experimental.pallas import tpu_sc as plsc

BM = 1024
KT = 1024
SC_CH = 256


def _codes_body(r_ref, rsq_ref, csq_ref, cbm2_ref, codes_ref):
    K = cbm2_ref.shape[0]
    nk = K // KT
    kidx = jax.lax.broadcasted_iota(jnp.int32, (BM, KT), 1).astype(jnp.float32)
    residual = r_ref[...]
    r_sq = rsq_ref[...]
    run_min = jnp.full((BM, 1), jnp.inf, jnp.float32)
    run_arg = jnp.zeros((BM, 1), jnp.float32)
    for t in range(nk):
        cb_tile = cbm2_ref[t * KT:(t + 1) * KT, :]
        dots = jax.lax.dot_general(
            residual, cb_tile, (((1,), (1,)), ((), ())),
            preferred_element_type=jnp.float32)
        c_sq = csq_ref[0, t * KT:(t + 1) * KT]
        dists = (r_sq + dots) + c_sq[None, :]
        tmin = jnp.min(dists, axis=1, keepdims=True)
        targ = jnp.min(jnp.where(dists == tmin, kidx, jnp.float32(K)),
                       axis=1, keepdims=True) + jnp.float32(t * KT)
        upd = tmin < run_min
        run_min = jnp.where(upd, tmin, run_min)
        run_arg = jnp.where(upd, targ, run_arg)
    codes_ref[...] = run_arg.astype(jnp.int32)


def _codes_call(B, K, D):
    return pl.pallas_call(
        _codes_body,
        grid=(B // BM,),
        in_specs=[
            pl.BlockSpec((BM, D), lambda i: (i, 0)),
            pl.BlockSpec((BM, 1), lambda i: (i, 0)),
            pl.BlockSpec((1, K), lambda i: (0, 0)),
            pl.BlockSpec((K, D), lambda i: (0, 0)),
        ],
        out_specs=pl.BlockSpec((BM, 1), lambda i: (i, 0)),
        out_shape=jax.ShapeDtypeStruct((B, 1), jnp.int32),
        compiler_params=pltpu.CompilerParams(
            dimension_semantics=("arbitrary",),
        ),
    )


def _sc_gather(cb, codes):
    B = codes.shape[0]
    K, D = cb.shape
    info = plsc.get_sparse_core_info()
    nw = info.num_cores * info.num_subcores
    b_per_w = B // nw
    mesh = plsc.VectorSubcoreMesh(core_axis_name="c", subcore_axis_name="s")

    @functools.partial(
        pl.kernel, mesh=mesh,
        out_type=jax.ShapeDtypeStruct((B, D), jnp.float32),
        scratch_types=[
            pltpu.VMEM((SC_CH,), jnp.int32),
            pltpu.VMEM((SC_CH, D), jnp.float32),
            pltpu.SemaphoreType.DMA,
        ],
    )
    def k(table_hbm, idx_hbm, out_hbm, idx_v, rows_v, sem):
        wid = lax.axis_index("s") * info.num_cores + lax.axis_index("c")
        base = wid * b_per_w
        for c in range(b_per_w // SC_CH):
            off = base + c * SC_CH
            pltpu.sync_copy(idx_hbm.at[pl.ds(off, SC_CH)], idx_v)
            pltpu.async_copy(table_hbm.at[idx_v], rows_v, sem).wait()
            pltpu.sync_copy(rows_v, out_hbm.at[pl.ds(off, SC_CH)])

    return k(cb, codes)


def _chain(x, cbs, cbm2s, csqs):
    B, D = x.shape
    K = cbs[0].shape[0]
    codes_call = _codes_call(B, K, D)
    r = x
    rsq = jnp.sum(x * x, axis=1, keepdims=True)
    codes = []
    qs = []
    for l in range(4):
        cl = codes_call(r, rsq, csqs[l], cbm2s[l])
        codes.append(cl)
        q = _sc_gather(cbs[l], cl.reshape(B))
        qs.append(q)
        if l < 3:
            r = r - q
            rsq = jnp.sum(r * r, axis=1, keepdims=True)
    quantized = ((qs[0] + qs[1]) + qs[2]) + qs[3]
    return jnp.concatenate(codes, axis=1), quantized


@jax.jit
def kernel(x, codebook0, codebook1, codebook2, codebook3):
    B, D = x.shape
    K = codebook0.shape[0]
    cbs = (codebook0, codebook1, codebook2, codebook3)
    csqs = [jnp.sum(cb * cb, axis=1).reshape(1, K) for cb in cbs]
    cbm2s = [-2.0 * cb for cb in cbs]
    h = B // 2
    c0, q0 = _chain(x[:h], cbs, cbm2s, csqs)
    c1, q1 = _chain(x[h:], cbs, cbm2s, csqs)
    return (jnp.concatenate([c0, c1], axis=0),
            jnp.concatenate([q0, q1], axis=0))


# speedup vs baseline: 1.0451x; 1.0451x over previous
import functools

import jax
import jax.numpy as jnp
from jax import lax
from jax.experimental import pallas as pl
from jax.experimental.pallas import tpu as pltpu
from jax.experimental.pallas import tpu_sc as plsc

BM = 1024
KT = 1024
SC_CH = 256


def _codes_body(r_ref, rsq_ref, csq_ref, cbm2_ref, codes_ref):
    K = cbm2_ref.shape[0]
    nk = K // KT
    kidx = jax.lax.broadcasted_iota(jnp.int32, (BM, KT), 1).astype(jnp.float32)
    residual = r_ref[...]
    r_sq = rsq_ref[...]
    run_min = jnp.full((BM, 1), jnp.inf, jnp.float32)
    run_arg = jnp.zeros((BM, 1), jnp.float32)
    for t in range(nk):
        cb_tile = cbm2_ref[t * KT:(t + 1) * KT, :]
        dots = jax.lax.dot_general(
            residual, cb_tile, (((1,), (1,)), ((), ())),
            preferred_element_type=jnp.float32)
        c_sq = csq_ref[0, t * KT:(t + 1) * KT]
        dists = (r_sq + dots) + c_sq[None, :]
        tmin = jnp.min(dists, axis=1, keepdims=True)
        targ = jnp.min(jnp.where(dists == tmin, kidx, jnp.float32(K)),
                       axis=1, keepdims=True) + jnp.float32(t * KT)
        upd = tmin < run_min
        run_min = jnp.where(upd, tmin, run_min)
        run_arg = jnp.where(upd, targ, run_arg)
    codes_ref[...] = run_arg.astype(jnp.int32)


def _codes_call(B, K, D):
    return pl.pallas_call(
        _codes_body,
        grid=(B // BM,),
        in_specs=[
            pl.BlockSpec((BM, D), lambda i: (i, 0)),
            pl.BlockSpec((BM, 1), lambda i: (i, 0)),
            pl.BlockSpec((1, K), lambda i: (0, 0)),
            pl.BlockSpec((K, D), lambda i: (0, 0)),
        ],
        out_specs=pl.BlockSpec((BM, 1), lambda i: (i, 0)),
        out_shape=jax.ShapeDtypeStruct((B, 1), jnp.int32),
        compiler_params=pltpu.CompilerParams(
            dimension_semantics=("arbitrary",),
        ),
    )


def _sc_gather(cb, codes):
    B = codes.shape[0]
    K, D = cb.shape
    info = plsc.get_sparse_core_info()
    nw = info.num_cores * info.num_subcores
    b_per_w = B // nw
    mesh = plsc.VectorSubcoreMesh(core_axis_name="c", subcore_axis_name="s")

    @functools.partial(
        pl.kernel, mesh=mesh,
        out_type=jax.ShapeDtypeStruct((B, D), jnp.float32),
        scratch_types=[
            pltpu.VMEM((SC_CH,), jnp.int32),
            pltpu.VMEM((SC_CH, D), jnp.float32),
            pltpu.SemaphoreType.DMA,
        ],
    )
    def k(table_hbm, idx_hbm, out_hbm, idx_v, rows_v, sem):
        wid = lax.axis_index("s") * info.num_cores + lax.axis_index("c")
        base = wid * b_per_w
        for c in range(b_per_w // SC_CH):
            off = base + c * SC_CH
            pltpu.sync_copy(idx_hbm.at[pl.ds(off, SC_CH)], idx_v)
            pltpu.async_copy(table_hbm.at[idx_v], rows_v, sem).wait()
            pltpu.sync_copy(rows_v, out_hbm.at[pl.ds(off, SC_CH)])

    return k(cb, codes)


def _chain(x, cbs, cbm2s, csqs):
    B, D = x.shape
    K = cbs[0].shape[0]
    codes_call = _codes_call(B, K, D)
    r = x
    rsq = jnp.sum(x * x, axis=1, keepdims=True)
    codes = []
    qs = []
    for l in range(4):
        cl = codes_call(r, rsq, csqs[l], cbm2s[l])
        codes.append(cl)
        q = _sc_gather(cbs[l], cl.reshape(B))
        qs.append(q)
        if l < 3:
            r = r - q
            rsq = jnp.sum(r * r, axis=1, keepdims=True)
    quantized = ((qs[0] + qs[1]) + qs[2]) + qs[3]
    return jnp.concatenate(codes, axis=1), quantized


@jax.jit
def kernel(x, codebook0, codebook1, codebook2, codebook3):
    B, D = x.shape
    K = codebook0.shape[0]
    cbs = (codebook0, codebook1, codebook2, codebook3)
    csqs = [jnp.sum(cb * cb, axis=1).reshape(1, K) for cb in cbs]
    cbm2s = [-2.0 * cb for cb in cbs]
    return _chain(x, cbs, cbm2s, csqs)
